# baseline (speedup 1.0000x reference)
import functools

import jax
import jax.numpy as jnp
from jax import lax
from jax.experimental import pallas as pl
from jax.experimental.pallas import tpu as pltpu
from jax.experimental.pallas import tpu_sc as plsc

NC = 2
NS = 16
NW = NC * NS
CHUNK = 128
RING = 16
K = 8

_MESH = plsc.VectorSubcoreMesh(core_axis_name="c", subcore_axis_name="s")
_SC_PARAMS = pltpu.CompilerParams(use_tc_tiling_on_sc=False)


def _sc_degree(col2d, zeros1):
    nch = col2d.shape[0] // NW
    n_pad = zeros1.shape[0]

    @functools.partial(
        pl.kernel,
        out_type=jax.ShapeDtypeStruct((NC, n_pad), jnp.float32),
        mesh=_MESH,
        scratch_types=[
            pltpu.VMEM((nch, CHUNK), jnp.int32),
            pltpu.VMEM((CHUNK,), jnp.float32),
            pltpu.VMEM_SHARED((n_pad,), jnp.float32),
            pltpu.SemaphoreType.DMA,
        ],
        compiler_params=_SC_PARAMS,
    )
    def k(col_hbm, zeros_hbm, deg_hbm, cidx, ones_f, acc_sh, dsem):
        c = lax.axis_index("c")
        s = lax.axis_index("s")
        wid = s * NC + c

        @pl.loop(0, CHUNK, step=16)
        def _(i):
            ones_f[pl.ds(i, 16)] = jnp.full((16,), 1.0, jnp.float32)

        pltpu.sync_copy(col_hbm.at[pl.ds(wid * nch, nch)], cidx)

        @pl.when(s == 0)
        def _():
            pltpu.sync_copy(zeros_hbm, acc_sh)

        plsc.subcore_barrier()

        @pl.loop(0, nch)
        def _(j):
            pltpu.sync_copy(ones_f, acc_sh.at[cidx.at[j]], add=True)

        plsc.subcore_barrier()

        @pl.when(s == 0)
        def _():
            pltpu.sync_copy(acc_sh, deg_hbm.at[c])

    return k(col2d, zeros1)


def _sc_aggregate(g, row2d, col2d, zeros2):
    n, d = g.shape
    nch = row2d.shape[0] // NW
    n_pad = zeros2.shape[0]
    irps = (n_pad // NS) // 8 * 8
    irlast = n_pad - irps * (NS - 1)
    wrps = (n // NS) // 8 * 8
    wrlast = n - wrps * (NS - 1)

    @functools.partial(
        pl.kernel,
        out_type=jax.ShapeDtypeStruct((NC, n, d), jnp.float32),
        mesh=_MESH,
        scratch_types=[
            pltpu.VMEM((nch, CHUNK), jnp.int32),
            pltpu.VMEM((nch, CHUNK), jnp.int32),
            pltpu.VMEM((RING, CHUNK, d), jnp.float32),
            pltpu.VMEM_SHARED((n_pad, d), jnp.float32),
            pltpu.VMEM_SHARED((n, d), jnp.float32),
            pltpu.SemaphoreType.DMA((K,)),
            pltpu.SemaphoreType.DMA((K,)),
        ],
        compiler_params=_SC_PARAMS,
    )
    def k(g_hbm, row_hbm, col_hbm, zeros_hbm, out_hbm,
          ridx, cidx, rows, acc_sh, g_sh, gsem, ssem):
        c = lax.axis_index("c")
        s = lax.axis_index("s")
        wid = s * NC + c

        pltpu.sync_copy(row_hbm.at[pl.ds(wid * nch, nch)], ridx)
        pltpu.sync_copy(col_hbm.at[pl.ds(wid * nch, nch)], cidx)

        @pl.when(s < NS - 1)
        def _():
            pltpu.sync_copy(zeros_hbm.at[pl.ds(s * irps, irps)],
                            acc_sh.at[pl.ds(s * irps, irps)])
            pltpu.sync_copy(g_hbm.at[pl.ds(s * wrps, wrps)],
                            g_sh.at[pl.ds(s * wrps, wrps)])

        @pl.when(s == NS - 1)
        def _():
            pltpu.sync_copy(zeros_hbm.at[pl.ds((NS - 1) * irps, irlast)],
                            acc_sh.at[pl.ds((NS - 1) * irps, irlast)])
            pltpu.sync_copy(g_hbm.at[pl.ds((NS - 1) * wrps, wrlast)],
                            g_sh.at[pl.ds((NS - 1) * wrps, wrlast)])

        plsc.subcore_barrier()

        @pl.loop(0, nch, step=K)
        def _(j):
            gh, sh = [], []
            for b in range(K):
                gh.append(pltpu.async_copy(
                    g_sh.at[ridx.at[j + b]], rows.at[b], gsem.at[b]))
            for b in range(K):
                gh[b].wait()
                sh.append(pltpu.async_copy(
                    rows.at[b], acc_sh.at[cidx.at[j + b]], ssem.at[b],
                    add=True))
            for b in range(K):
                sh[b].wait()

        plsc.subcore_barrier()

        @pl.when(s < NS - 1)
        def _():
            pltpu.sync_copy(acc_sh.at[pl.ds(s * wrps, wrps)],
                            out_hbm.at[c, pl.ds(s * wrps, wrps)])

        @pl.when(s == NS - 1)
        def _():
            pltpu.sync_copy(acc_sh.at[pl.ds((NS - 1) * wrps, wrlast)],
                            out_hbm.at[c, pl.ds((NS - 1) * wrps, wrlast)])

    return k(g, row2d, col2d, zeros2)


def _tc_lin1_scale(x, w1, b1r, degp_t):
    n = x.shape[0]
    d = w1.shape[1]

    def body(x_ref, w_ref, b_ref, dp_ref, dinv_ref, g_ref):
        hlin = jnp.dot(x_ref[...], w_ref[...],
                       preferred_element_type=jnp.float32) + b_ref[...]
        dp = dp_ref[...]
        deg = dp[:, 0:1] + dp[:, 1:2] + 1.0
        dinv = lax.rsqrt(deg)
        dinv_ref[...] = dinv
        g_ref[...] = dinv * hlin

    return pl.pallas_call(
        body,
        out_shape=[jax.ShapeDtypeStruct((n, 1), jnp.float32),
                   jax.ShapeDtypeStruct((n, d), jnp.float32)],
    )(x, w1, b1r, degp_t)


def _tc_layer2(accp, dinv, g1, w2p, b2r):
    n, d = g1.shape

    def body(a_ref, dinv_ref, g_ref, w_ref, b_ref, o_ref):
        arr = a_ref[...]
        dv = dinv_ref[...]
        h1 = jnp.maximum(dv * (arr[0] + arr[1] + g_ref[...]), 0.0)
        o_ref[...] = dv * (jnp.dot(h1, w_ref[...],
                                   preferred_element_type=jnp.float32)
                           + b_ref[...])

    return pl.pallas_call(
        body, out_shape=jax.ShapeDtypeStruct((n, d), jnp.float32),
    )(accp, dinv, g1, w2p, b2r)


def _tc_final(accp, dinv, g2, ncls):
    n, d = g2.shape

    def body(a_ref, dinv_ref, g_ref, o_ref):
        arr = a_ref[...]
        o = dinv_ref[...] * (arr[0] + arr[1] + g_ref[...])
        lane = lax.broadcasted_iota(jnp.int32, (1, d), 1)
        mask = lane < ncls
        m = jnp.max(jnp.where(mask, o, -jnp.inf), axis=1, keepdims=True)
        e = jnp.where(mask, jnp.exp(o - m), 0.0)
        lse = m + jnp.log(jnp.sum(e, axis=1, keepdims=True))
        o_ref[...] = (o - lse)[:, :ncls]

    return pl.pallas_call(
        body, out_shape=jax.ShapeDtypeStruct((n, ncls), jnp.float32),
    )(accp, dinv, g2)


def kernel(x, edge_index, W1, b1, W2, b2):
    n = x.shape[0]
    e = edge_index.shape[1]
    dh = W1.shape[1]
    ncls = W2.shape[1]

    grp = NW * CHUNK * RING
    epad = -(-e // grp) * grp
    row = edge_index[0]
    col = edge_index[1]
    rowp = jnp.concatenate(
        [row, jnp.zeros((epad - e,), row.dtype)]).reshape(-1, CHUNK)
    colp = jnp.concatenate(
        [col, jnp.full((epad - e,), n, col.dtype)]).reshape(-1, CHUNK)

    zeros1 = jnp.zeros((n + 8,), jnp.float32)
    zeros2 = jnp.zeros((n + 8, dh), jnp.float32)
    b1r = b1.reshape(1, dh)
    w2p = jnp.pad(W2, ((0, 0), (0, dh - ncls)))
    b2r = jnp.pad(b2, (0, dh - ncls)).reshape(1, dh)

    degp = _sc_degree(colp, zeros1)
    dinv, g1 = _tc_lin1_scale(x, W1, b1r, degp.T[:n])
    acc1 = _sc_aggregate(g1, rowp, colp, zeros2)
    g2 = _tc_layer2(acc1, dinv, g1, w2p, b2r)
    acc2 = _sc_aggregate(g2, rowp, colp, zeros2)
    return _tc_final(acc2, dinv, g2, ncls)


# speedup vs baseline: 1.0252x; 1.0252x over previous
import functools

import jax
import jax.numpy as jnp
from jax import lax
from jax.experimental import pallas as pl
from jax.experimental.pallas import tpu as pltpu
from jax.experimental.pallas import tpu_sc as plsc

NC = 2
NS = 16
NW = NC * NS
CHUNK = 128
RING = 16
K = 8

_MESH = plsc.VectorSubcoreMesh(core_axis_name="c", subcore_axis_name="s")
_SC_PARAMS = pltpu.CompilerParams(use_tc_tiling_on_sc=False)


def _sc_degree(col2d, zeros1):
    nch = col2d.shape[0] // NW
    n_pad = zeros1.shape[0]

    @functools.partial(
        pl.kernel,
        out_type=jax.ShapeDtypeStruct((NC, n_pad), jnp.float32),
        mesh=_MESH,
        scratch_types=[
            pltpu.VMEM((nch, CHUNK), jnp.int32),
            pltpu.VMEM((CHUNK,), jnp.float32),
            pltpu.VMEM_SHARED((n_pad,), jnp.float32),
            pltpu.SemaphoreType.DMA,
        ],
        compiler_params=_SC_PARAMS,
    )
    def k(col_hbm, zeros_hbm, deg_hbm, cidx, ones_f, acc_sh, dsem):
        c = lax.axis_index("c")
        s = lax.axis_index("s")
        wid = s * NC + c

        @pl.loop(0, CHUNK, step=16)
        def _(i):
            ones_f[pl.ds(i, 16)] = jnp.full((16,), 1.0, jnp.float32)

        pltpu.sync_copy(col_hbm.at[pl.ds(wid * nch, nch)], cidx)

        @pl.when(s == 0)
        def _():
            pltpu.sync_copy(zeros_hbm, acc_sh)

        plsc.subcore_barrier()

        @pl.loop(0, nch)
        def _(j):
            pltpu.sync_copy(ones_f, acc_sh.at[cidx.at[j]], add=True)

        plsc.subcore_barrier()

        @pl.when(s == 0)
        def _():
            pltpu.sync_copy(acc_sh, deg_hbm.at[c])

    return k(col2d, zeros1)


def _sc_aggregate(g, row2d, col2d, zeros2):
    n, d = g.shape
    nch = row2d.shape[0] // NW
    n_pad = zeros2.shape[0]
    irps = (n_pad // NS) // 8 * 8
    irlast = n_pad - irps * (NS - 1)
    wrps = (n // NS) // 8 * 8
    wrlast = n - wrps * (NS - 1)

    @functools.partial(
        pl.kernel,
        out_type=jax.ShapeDtypeStruct((NC, n, d), jnp.float32),
        mesh=_MESH,
        scratch_types=[
            pltpu.VMEM((nch, CHUNK), jnp.int32),
            pltpu.VMEM((nch, CHUNK), jnp.int32),
            pltpu.VMEM((RING, CHUNK, d), jnp.float32),
            pltpu.VMEM_SHARED((n_pad, d), jnp.float32),
            pltpu.VMEM_SHARED((n, d), jnp.float32),
            pltpu.SemaphoreType.DMA((K,)),
            pltpu.SemaphoreType.DMA((K,)),
        ],
        compiler_params=_SC_PARAMS,
    )
    def k(g_hbm, row_hbm, col_hbm, zeros_hbm, out_hbm,
          ridx, cidx, rows, acc_sh, g_sh, gsem, ssem):
        c = lax.axis_index("c")
        s = lax.axis_index("s")
        wid = s * NC + c

        ph = [pltpu.async_copy(row_hbm.at[pl.ds(wid * nch, nch)], ridx,
                               gsem.at[0]),
              pltpu.async_copy(col_hbm.at[pl.ds(wid * nch, nch)], cidx,
                               gsem.at[1])]

        @pl.when(s < NS - 1)
        def _():
            h1 = pltpu.async_copy(zeros_hbm.at[pl.ds(s * irps, irps)],
                                  acc_sh.at[pl.ds(s * irps, irps)],
                                  gsem.at[2])
            h2 = pltpu.async_copy(g_hbm.at[pl.ds(s * wrps, wrps)],
                                  g_sh.at[pl.ds(s * wrps, wrps)],
                                  gsem.at[3])
            h1.wait()
            h2.wait()

        @pl.when(s == NS - 1)
        def _():
            h1 = pltpu.async_copy(zeros_hbm.at[pl.ds((NS - 1) * irps, irlast)],
                                  acc_sh.at[pl.ds((NS - 1) * irps, irlast)],
                                  gsem.at[2])
            h2 = pltpu.async_copy(g_hbm.at[pl.ds((NS - 1) * wrps, wrlast)],
                                  g_sh.at[pl.ds((NS - 1) * wrps, wrlast)],
                                  gsem.at[3])
            h1.wait()
            h2.wait()

        for h in ph:
            h.wait()
        plsc.subcore_barrier()

        @pl.loop(0, nch, step=K)
        def _(j):
            gh, sh = [], []
            for b in range(K):
                gh.append(pltpu.async_copy(
                    g_sh.at[ridx.at[j + b]], rows.at[b], gsem.at[b]))
            for b in range(K):
                gh[b].wait()
                sh.append(pltpu.async_copy(
                    rows.at[b], acc_sh.at[cidx.at[j + b]], ssem.at[b],
                    add=True))
            for b in range(K):
                sh[b].wait()

        plsc.subcore_barrier()

        @pl.when(s < NS - 1)
        def _():
            pltpu.sync_copy(acc_sh.at[pl.ds(s * wrps, wrps)],
                            out_hbm.at[c, pl.ds(s * wrps, wrps)])

        @pl.when(s == NS - 1)
        def _():
            pltpu.sync_copy(acc_sh.at[pl.ds((NS - 1) * wrps, wrlast)],
                            out_hbm.at[c, pl.ds((NS - 1) * wrps, wrlast)])

    return k(g, row2d, col2d, zeros2)


def _tc_lin1_scale(x, w1, b1r, degp_t):
    n = x.shape[0]
    d = w1.shape[1]

    def body(x_ref, w_ref, b_ref, dp_ref, dinv_ref, g_ref):
        hlin = jnp.dot(x_ref[...], w_ref[...],
                       preferred_element_type=jnp.float32) + b_ref[...]
        dp = dp_ref[...]
        deg = dp[:, 0:1] + dp[:, 1:2] + 1.0
        dinv = lax.rsqrt(deg)
        dinv_ref[...] = dinv
        g_ref[...] = dinv * hlin

    return pl.pallas_call(
        body,
        out_shape=[jax.ShapeDtypeStruct((n, 1), jnp.float32),
                   jax.ShapeDtypeStruct((n, d), jnp.float32)],
    )(x, w1, b1r, degp_t)


def _tc_layer2(accp, dinv, g1, w2p, b2r):
    n, d = g1.shape

    def body(a_ref, dinv_ref, g_ref, w_ref, b_ref, o_ref):
        arr = a_ref[...]
        dv = dinv_ref[...]
        h1 = jnp.maximum(dv * (arr[0] + arr[1] + g_ref[...]), 0.0)
        o_ref[...] = dv * (jnp.dot(h1, w_ref[...],
                                   preferred_element_type=jnp.float32)
                           + b_ref[...])

    return pl.pallas_call(
        body, out_shape=jax.ShapeDtypeStruct((n, d), jnp.float32),
    )(accp, dinv, g1, w2p, b2r)


def _tc_final(accp, dinv, g2, ncls):
    n, d = g2.shape

    def body(a_ref, dinv_ref, g_ref, o_ref):
        arr = a_ref[...]
        o = dinv_ref[...] * (arr[0] + arr[1] + g_ref[...])
        lane = lax.broadcasted_iota(jnp.int32, (1, d), 1)
        mask = lane < ncls
        m = jnp.max(jnp.where(mask, o, -jnp.inf), axis=1, keepdims=True)
        e = jnp.where(mask, jnp.exp(o - m), 0.0)
        lse = m + jnp.log(jnp.sum(e, axis=1, keepdims=True))
        o_ref[...] = (o - lse)[:, :ncls]

    return pl.pallas_call(
        body, out_shape=jax.ShapeDtypeStruct((n, ncls), jnp.float32),
    )(accp, dinv, g2)


def kernel(x, edge_index, W1, b1, W2, b2):
    n = x.shape[0]
    e = edge_index.shape[1]
    dh = W1.shape[1]
    ncls = W2.shape[1]

    grp = NW * CHUNK * RING
    epad = -(-e // grp) * grp
    row = edge_index[0]
    col = edge_index[1]
    rowp = jnp.concatenate(
        [row, jnp.zeros((epad - e,), row.dtype)]).reshape(-1, CHUNK)
    colp = jnp.concatenate(
        [col, jnp.full((epad - e,), n, col.dtype)]).reshape(-1, CHUNK)

    zeros1 = jnp.zeros((n + 8,), jnp.float32)
    zeros2 = jnp.zeros((n + 8, dh), jnp.float32)
    b1r = b1.reshape(1, dh)
    w2p = jnp.pad(W2, ((0, 0), (0, dh - ncls)))
    b2r = jnp.pad(b2, (0, dh - ncls)).reshape(1, dh)

    degp = _sc_degree(colp, zeros1)
    dinv, g1 = _tc_lin1_scale(x, W1, b1r, degp.T[:n])
    acc1 = _sc_aggregate(g1, rowp, colp, zeros2)
    g2 = _tc_layer2(acc1, dinv, g1, w2p, b2r)
    acc2 = _sc_aggregate(g2, rowp, colp, zeros2)
    return _tc_final(acc2, dinv, g2, ncls)
